# baseline (speedup 1.0000x reference)
import functools

import jax
import jax.numpy as jnp
from jax import lax
from jax.experimental import pallas as pl
from jax.experimental.pallas import tpu as pltpu
from jax.experimental.pallas import tpu_sc as plsc
from jax._src.pallas import mpmd

NC = 2
NS = 16
NW = NC * NS
K = 8
NBUF = 4
RSCS = 512
SK = 8


def _tec_body(n_tec):
    def tec_main(ids_hbm, table_hbm, out_hbm, idx_v, rows_v, sidx, sbuf,
             gsem0, gsem1, gsem2, gsem3, sg0, sg1, ss0, ss1):
        del sidx, sbuf, sg0, sg1, ss0, ss1
        tpw = n_tec // NW
        cpw = tpw // K
        wid = lax.axis_index("s") * NC + lax.axis_index("c")
        chunk0 = wid * cpw
        pltpu.sync_copy(ids_hbm.at[pl.ds(wid * tpw, tpw)], idx_v)
        gsems = (gsem0, gsem1, gsem2, gsem3)

        def gather(g, b):
            pltpu.async_copy(
                table_hbm.at[idx_v.at[pl.ds(g * K, K)]], rows_v.at[b],
                gsems[b])

        def wait_gather(b):
            pltpu.make_async_copy(
                table_hbm.at[idx_v.at[pl.ds(0, K)]], rows_v.at[b],
                gsems[b]).wait()

        def scatter(g, b):
            pltpu.sync_copy(
                rows_v.at[b], out_hbm.at[pl.ds((chunk0 + g) * K, K)])

        for b in range(NBUF):
            gather(b, b)

        def step(h, _):
            for b in range(NBUF):
                j = h * NBUF + b
                wait_gather(b)
                scatter(j, b)
                gather(j + NBUF, b)
            return 0

        lax.fori_loop(0, (cpw - NBUF) // NBUF, step, 0)

        for j in range(cpw - NBUF, cpw):
            b = j % NBUF
            wait_gather(b)
            scatter(j, b)

    return tec_main


def _scs_body(n_tec):
    def scs_main(ids_hbm, table_hbm, out_hbm, idx_v, rows_v, sidx, sbuf,
             gsem0, gsem1, gsem2, gsem3, sg0, sg1, ss0, ss1):
        del idx_v, rows_v, gsem0, gsem1, gsem2, gsem3
        c = lax.axis_index("c")
        base = n_tec + c * RSCS
        pltpu.sync_copy(ids_hbm.at[pl.ds(base, RSCS)], sidx)
        sgs = (sg0, sg1)
        sss = (ss0, ss1)
        nchunk = RSCS // SK

        def gather_rows(j, b):
            for r in range(SK):
                i = sidx[j * SK + r]
                pltpu.async_copy(
                    table_hbm.at[pl.ds(i, 1)], sbuf.at[b, pl.ds(r, 1)],
                    sgs[b])

        def wait_gathers(b):
            for _ in range(SK):
                pltpu.make_async_copy(
                    table_hbm.at[pl.ds(0, 1)], sbuf.at[b, pl.ds(0, 1)],
                    sgs[b]).wait()

        def out_copy(j, b):
            pltpu.async_copy(
                sbuf.at[b], out_hbm.at[pl.ds(base + j * SK, SK)], sss[b])

        def wait_out(b):
            pltpu.make_async_copy(
                sbuf.at[0], out_hbm.at[pl.ds(base, SK)], sss[b]).wait()

        gather_rows(0, 0)
        gather_rows(1, 1)

        def step(h, _):
            for b in range(2):
                j = h * 2 + b
                wait_gathers(b)
                out_copy(j, b)
                wait_out(b)
                gather_rows(j + 2, b)
            return 0

        lax.fori_loop(0, (nchunk - 2) // 2, step, 0)

        for j in range(nchunk - 2, nchunk):
            b = j % 2
            wait_gathers(b)
            out_copy(j, b)
            wait_out(b)

    return scs_main


def _make_emb(n_tokens, vocab, d_model):
    vmesh = plsc.VectorSubcoreMesh(core_axis_name="c", subcore_axis_name="s")
    smesh = plsc.ScalarSubcoreMesh(axis_name="c")
    n_tec = n_tokens - NC * RSCS
    return mpmd.mpmd_map(
        [(smesh, _scs_body(n_tec)), (vmesh, _tec_body(n_tec))],
        out_types=jax.ShapeDtypeStruct((n_tokens, d_model), jnp.float32),
        scratch_types=[
            (pltpu.VMEM @ vmesh)((n_tec // NW,), jnp.int32),
            (pltpu.VMEM @ vmesh)((NBUF, K, d_model), jnp.float32),
            (pltpu.SMEM @ smesh)((RSCS,), jnp.int32),
            pltpu.VMEM_SHARED((2, SK, d_model), jnp.float32),
        ]
        + [pltpu.SemaphoreType.DMA @ vmesh] * NBUF
        + [pltpu.SemaphoreType.DMA @ smesh] * 4,
    )


def kernel(input_ids, attention_mask, labels, weight):
    b, s = input_ids.shape
    vocab, d_model = weight.shape
    ids_flat = input_ids.reshape(-1).astype(jnp.int32)
    out = _make_emb(b * s, vocab, d_model)(ids_flat, weight)
    hidden_states = out.reshape(b, s, d_model)
    position_ids = jnp.arange(s, dtype=jnp.int32)[None, :]
    return (hidden_states, attention_mask, position_ids, labels)


# speedup vs baseline: 1.0285x; 1.0285x over previous
import functools

import jax
import jax.numpy as jnp
from jax import lax
from jax.experimental import pallas as pl
from jax.experimental.pallas import tpu as pltpu
from jax.experimental.pallas import tpu_sc as plsc

NC = 2
NS = 16
NW = NC * NS
K = 8


NBUF = 4


def _emb_body(ids_hbm, table_hbm, out_hbm, idx_v, rows_v,
              gsem0, gsem1, gsem2, gsem3):
    tpw = ids_hbm.shape[0] // NW
    cpw = tpw // K
    wid = lax.axis_index("s") * NC + lax.axis_index("c")
    chunk0 = wid * cpw
    pltpu.sync_copy(ids_hbm.at[pl.ds(wid * tpw, tpw)], idx_v)
    gsems = (gsem0, gsem1, gsem2, gsem3)

    def gather(g, b):
        pltpu.async_copy(table_hbm.at[idx_v.at[pl.ds(g * K, K)]], rows_v.at[b], gsems[b])

    def wait_gather(b):
        pltpu.make_async_copy(
            table_hbm.at[idx_v.at[pl.ds(0, K)]], rows_v.at[b], gsems[b]).wait()

    def scatter(g, b):
        pltpu.sync_copy(rows_v.at[b], out_hbm.at[pl.ds((chunk0 + g) * K, K)])

    for b in range(NBUF):
        gather(b, b)

    def step(h, _):
        for b in range(NBUF):
            j = h * NBUF + b
            wait_gather(b)
            scatter(j, b)
            gather(j + NBUF, b)
        return 0

    lax.fori_loop(0, (cpw - NBUF) // NBUF, step, 0)

    for j in range(cpw - NBUF, cpw):
        b = j % NBUF
        wait_gather(b)
        scatter(j, b)


def _make_emb(n_tokens, vocab, d_model):
    mesh = plsc.VectorSubcoreMesh(core_axis_name="c", subcore_axis_name="s")
    return functools.partial(
        pl.kernel,
        mesh=mesh,
        out_type=jax.ShapeDtypeStruct((n_tokens, d_model), jnp.float32),
        scratch_types=[
            pltpu.VMEM((n_tokens // NW,), jnp.int32),
            pltpu.VMEM((NBUF, K, d_model), jnp.float32),
        ] + [pltpu.SemaphoreType.DMA] * NBUF,
    )(_emb_body)


def kernel(input_ids, attention_mask, labels, weight):
    b, s = input_ids.shape
    vocab, d_model = weight.shape
    ids_flat = input_ids.reshape(-1).astype(jnp.int32)
    out = _make_emb(b * s, vocab, d_model)(ids_flat, weight)
    hidden_states = out.reshape(b, s, d_model)
    position_ids = jnp.arange(s, dtype=jnp.int32)[None, :]
    return (hidden_states, attention_mask, position_ids, labels)
